# baseline (speedup 1.0000x reference)
import functools

import jax
import jax.numpy as jnp
from jax import lax
from jax.experimental import pallas as pl
from jax.experimental.pallas import tpu as pltpu
from jax.experimental.pallas import tpu_sc as plsc

NC = 2
NS = 16
NW = NC * NS
L = 16
K = 128
H = 64
QK = 64


def _make_deg_kernel(n_pad, rt, qrt):
    @functools.partial(
        pl.kernel,
        out_type=(
            jax.ShapeDtypeStruct((NW, n_pad), jnp.float32),
            jax.ShapeDtypeStruct((NW, qrt, K), jnp.float32),
            jax.ShapeDtypeStruct((NW, qrt, K), jnp.float32),
        ),
        mesh=plsc.VectorSubcoreMesh(core_axis_name="c", subcore_axis_name="s"),
        scratch_types=[
            pltpu.VMEM((n_pad,), jnp.float32),
            pltpu.VMEM((rt, K), jnp.int32),
            pltpu.VMEM((qrt, K), jnp.int32),
            pltpu.VMEM((qrt, K), jnp.int32),
            pltpu.VMEM((n_pad,), jnp.int32),
            pltpu.VMEM((qrt, K), jnp.float32),
            pltpu.VMEM((qrt, K), jnp.float32),
        ],
        compiler_params=pltpu.CompilerParams(needs_layout_passes=False),
    )
    def deg_kernel(dstr, qsr, qdr, pkp, outd, idiff, spart,
                   deg_v, idx_v, qi_s, qi_d, pk_v, fb1, fb2):
        cid = lax.axis_index("c")
        sid = lax.axis_index("s")
        wid = cid * NS + sid
        ones = jnp.ones((L,), jnp.float32)
        pltpu.sync_copy(dstr.at[pl.ds(wid * rt, rt)], idx_v)
        pltpu.sync_copy(qsr.at[wid], qi_s)
        pltpu.sync_copy(qdr.at[wid], qi_d)
        pltpu.sync_copy(pkp, pk_v)

        def zero_body(i, _):
            deg_v[pl.ds(i * L, L)] = jnp.zeros((L,), jnp.float32)
            return 0
        lax.fori_loop(0, n_pad // L, zero_body, 0)

        def row_body(j, _):
            for g in range(K // L):
                dv = idx_v[j, pl.ds(g * L, L)]
                plsc.addupdate_scatter(deg_v, [dv], ones)
            return 0
        lax.fori_loop(0, rt, row_body, 0)
        pltpu.sync_copy(deg_v, outd.at[wid])

        def feat_body(j, _):
            for g in range(K // L):
                qsv = qi_s[j, pl.ds(g * L, L)]
                qdv = qi_d[j, pl.ds(g * L, L)]
                pvs = plsc.load_gather(pk_v, [qsv])
                pvd = plsc.load_gather(pk_v, [qdv])
                fb1[j, pl.ds(g * L, L)] = (
                    (pvd >> 7) - (pvs >> 7)).astype(jnp.float32)
                fb2[j, pl.ds(g * L, L)] = jnp.where(
                    (pvd & 127) == (pvs & 127),
                    jnp.float32(1.0), jnp.float32(0.0))
            return 0
        lax.fori_loop(0, qrt, feat_body, 0)
        pltpu.sync_copy(fb1, idiff.at[wid])
        pltpu.sync_copy(fb2, spart.at[wid])

    return deg_kernel


def _make_scatter_kernel(n_pad, rt):
    zr = n_pad // NS
    hrt = rt // 2

    @functools.partial(
        pl.kernel,
        out_type=jax.ShapeDtypeStruct((NC, n_pad, 128), jnp.float32),
        mesh=plsc.VectorSubcoreMesh(core_axis_name="c", subcore_axis_name="s"),
        scratch_types=[
            pltpu.VMEM_SHARED((n_pad, 128), jnp.float32),
            pltpu.VMEM((hrt, K), jnp.int32),
            pltpu.VMEM((hrt, K), jnp.int32),
            pltpu.VMEM((K, 128), jnp.float32),
            pltpu.VMEM((K, 128), jnp.float32),
            pltpu.VMEM((8, 128), jnp.float32),
            pltpu.SemaphoreType.DMA,
            pltpu.SemaphoreType.DMA,
        ],
    )
    def scatter_kernel(hws_h, srcr, dstr, accp_h,
                       asp, idx_s, idx_d, rows0, rows1, zbuf,
                       sem0, sem1):
        cid = lax.axis_index("c")
        sid = lax.axis_index("s")
        wid = cid * NS + sid

        def zb(i, _):
            for g in range(128 // L):
                zbuf[i, pl.ds(g * L, L)] = jnp.zeros((L,), jnp.float32)
            return 0
        lax.fori_loop(0, 8, zb, 0)

        if True:
            def zc(t, _):
                pltpu.sync_copy(zbuf, asp.at[pl.ds(sid * zr + t * 8, 8)])
                return 0
            lax.fori_loop(0, zr // 8, zc, 0)
            plsc.subcore_barrier()

            for h in range(2):
                pltpu.sync_copy(srcr.at[pl.ds(wid * rt + h * hrt, hrt)], idx_s)
                pltpu.sync_copy(dstr.at[pl.ds(wid * rt + h * hrt, hrt)], idx_d)
                def row_body(j, _):
                    pltpu.async_copy(hws_h.at[idx_s.at[j]], rows0, sem0).wait()
                    pltpu.sync_copy(rows0, asp.at[idx_d.at[j]], add=True)
                    return 0
                lax.fori_loop(0, hrt, row_body, 0)
            plsc.subcore_barrier()

            pltpu.sync_copy(asp.at[pl.ds(sid * zr, zr)],
                            accp_h.at[cid, pl.ds(sid * zr, zr)])

    return scatter_kernel


def _make_query_kernel(n_pad, q_pad, qrt):
    zr = n_pad // NS

    @functools.partial(
        pl.kernel,
        out_type=jax.ShapeDtypeStruct((q_pad, 128), jnp.float32),
        mesh=plsc.VectorSubcoreMesh(core_axis_name="c", subcore_axis_name="s"),
        scratch_types=[
            pltpu.VMEM((qrt, K), jnp.int32),
            pltpu.VMEM((qrt, K), jnp.int32),
            pltpu.VMEM((K, 128), jnp.float32),
            pltpu.VMEM((K, 128), jnp.float32),
            pltpu.VMEM((K, 128), jnp.float32),
            pltpu.VMEM((K, 128), jnp.float32),
            pltpu.SemaphoreType.DMA,
            pltpu.SemaphoreType.DMA,
            pltpu.SemaphoreType.DMA,
            pltpu.SemaphoreType.DMA,
        ],
    )
    def query_kernel(u_h, v_h, qsr, qdr, t1_h,
                     idx_s, idx_d, rs0, rs1, rd0, rd1,
                     ss0, ss1, sd0, sd1):
        cid = lax.axis_index("c")
        sid = lax.axis_index("s")
        wid = cid * NS + sid
        pltpu.sync_copy(qsr.at[wid], idx_s)
        pltpu.sync_copy(qdr.at[wid], idx_d)

        def isl(ib, j):
            return ib.at[j]

        def addbuf(db, sb):
            def ab(i, _):
                for g in range(128 // L):
                    sl = pl.ds(g * L, L)
                    db[i, sl] = db[i, sl] + sb[i, sl]
                return 0
            lax.fori_loop(0, K, ab, 0)

        if True:
            def row_body(j, _):
                cu = pltpu.async_copy(u_h.at[isl(idx_s, j)], rs0, ss0)
                cv = pltpu.async_copy(v_h.at[isl(idx_d, j)], rd0, sd0)
                cu.wait()
                cv.wait()
                addbuf(rs0, rd0)
                pltpu.sync_copy(rs0, t1_h.at[pl.ds((wid * qrt + j) * K, K)])
                return 0
            lax.fori_loop(0, qrt, row_body, 0)

    return query_kernel


def _tc_first(degp_blk, x_blk, w_blk, dinv_blk, hws_blk):
    deg = jnp.sum(degp_blk[...], axis=0) + 1.0
    dinv = lax.rsqrt(deg)
    dinv_blk[...] = dinv
    hw = jnp.dot(x_blk[...], w_blk[...], preferred_element_type=jnp.float32)
    hws_blk[...] = hw * dinv[:, None]


def _tc_mid(accp_blk, hws_blk, dinv_blk, b_blk, w_blk, out_blk):
    acc = accp_blk[0] + accp_blk[1] + hws_blk[...]
    dinv = dinv_blk[...]
    h = jax.nn.relu(acc * dinv[:, None] + b_blk[...][None, :])
    out_blk[...] = jnp.dot(h, w_blk[...],
                           preferred_element_type=jnp.float32) * dinv[:, None]


def _tc_last(accp_blk, hws_blk, dinv_blk, b_blk, w_blk, bf_blk,
             w1a_blk, w1b_blk, u_blk, v_blk):
    acc = accp_blk[0] + accp_blk[1] + hws_blk[...]
    dinv = dinv_blk[...]
    h = jax.nn.relu(acc * dinv[:, None] + b_blk[...][None, :])
    z = jnp.dot(h, w_blk[...],
                preferred_element_type=jnp.float32) + bf_blk[...][None, :]
    u_blk[...] = jnp.dot(z, w1a_blk[...], preferred_element_type=jnp.float32)
    v_blk[...] = jnp.dot(z, w1b_blk[...], preferred_element_type=jnp.float32)


def _tc_decoder(t1_blk, idf_blk, spt_blk, wci, wcs, b1,
                w2, b2, w3, b3, out_blk):
    t = t1_blk[...]
    t = t + idf_blk[...][:, None] * wci[...][None, :]
    t = t + spt_blk[...][:, None] * wcs[...][None, :]
    t = jax.nn.relu(t + b1[...][None, :])
    t = jax.nn.relu(jnp.dot(t, w2[...], preferred_element_type=jnp.float32)
                    + b2[...][None, :])
    logit = jnp.sum(t * w3[...][None, :], axis=1) + b3[0, 0]
    out_blk[...] = jax.nn.sigmoid(logit)


def kernel(x, edge_index, query_edges, int_values, partition_ids,
           W1, b1, W2, b2, Wf, bf, Wd1, bd1, Wd2, bd2, Wd3, bd3):
    n, f_in = x.shape
    e = edge_index.shape[1]
    q = query_edges.shape[1]
    hid = W1.shape[1]

    bn = 2048
    n_pad = ((n + bn - 1) // bn) * bn
    n_blocks = n_pad // bn

    rt = -(-e // (K * NW))
    rt = ((rt + 3) // 4) * 4
    e_pad = rt * NW * K
    src_r = jnp.pad(edge_index[0], (0, e_pad - e)).reshape(rt * NW, K)
    dst_r = jnp.pad(edge_index[1], (0, e_pad - e),
                    constant_values=n).reshape(rt * NW, K)

    qrt = -(-q // (K * NW))
    qrt = qrt + (qrt % 2)
    q_pad = qrt * NW * K
    qs_r = jnp.pad(query_edges[0], (0, q_pad - q)).reshape(NW, qrt, K)
    qd_r = jnp.pad(query_edges[1], (0, q_pad - q)).reshape(NW, qrt, K)
    pk_p = jnp.pad(int_values * 128 + partition_ids, (0, n_pad - n))

    deg_kernel = _make_deg_kernel(n_pad, rt, qrt)
    scatter_kernel = _make_scatter_kernel(n_pad, rt)
    query_kernel = _make_query_kernel(n_pad, q_pad, qrt)

    degp, idiff, spart = deg_kernel(dst_r, qs_r, qd_r, pk_p)
    idiff = idiff.reshape(q_pad)
    spart = spart.reshape(q_pad)

    grid_n = (n_blocks,)
    full_spec = pl.BlockSpec((bn, hid), lambda i: (i, 0))
    full_shape = jax.ShapeDtypeStruct((n_pad, hid), jnp.float32)
    accp_spec = pl.BlockSpec((NC, bn, hid), lambda i: (0, i, 0))

    dinv, hws1 = pl.pallas_call(
        _tc_first,
        grid=grid_n,
        in_specs=[
            pl.BlockSpec((NW, bn), lambda i: (0, i)),
            pl.BlockSpec((bn, f_in), lambda i: (i, 0)),
            pl.BlockSpec((f_in, hid), lambda i: (0, 0)),
        ],
        out_specs=[pl.BlockSpec((bn,), lambda i: (i,)), full_spec],
        out_shape=[jax.ShapeDtypeStruct((n_pad,), jnp.float32), full_shape],
    )(degp, x, W1)

    acc1 = scatter_kernel(hws1, src_r, dst_r)

    hws2 = pl.pallas_call(
        _tc_mid,
        grid=grid_n,
        in_specs=[
            accp_spec,
            full_spec,
            pl.BlockSpec((bn,), lambda i: (i,)),
            pl.BlockSpec((hid,), lambda i: (0,)),
            pl.BlockSpec((hid, hid), lambda i: (0, 0)),
        ],
        out_specs=full_spec,
        out_shape=full_shape,
    )(acc1, hws1, dinv, b1, W2)

    acc2 = scatter_kernel(hws2, src_r, dst_r)

    emb = Wf.shape[1]
    hid2 = Wd2.shape[1]
    w1a = Wd1[:emb]
    w1b = Wd1[emb:2 * emb]
    wci = Wd1[2 * emb]
    wcs = Wd1[2 * emb + 1]
    w2p = jnp.pad(Wd2, ((0, 0), (0, hid - hid2)))
    b2p = jnp.pad(bd2, (0, hid - hid2))
    w3v = jnp.pad(Wd3[:, 0], (0, hid - hid2))
    b3 = bd3.reshape(1, 1)

    u, v = pl.pallas_call(
        _tc_last,
        grid=grid_n,
        in_specs=[
            accp_spec,
            full_spec,
            pl.BlockSpec((bn,), lambda i: (i,)),
            pl.BlockSpec((hid,), lambda i: (0,)),
            pl.BlockSpec((hid, emb), lambda i: (0, 0)),
            pl.BlockSpec((emb,), lambda i: (0,)),
            pl.BlockSpec((emb, hid), lambda i: (0, 0)),
            pl.BlockSpec((emb, hid), lambda i: (0, 0)),
        ],
        out_specs=[full_spec, full_spec],
        out_shape=[full_shape, full_shape],
    )(acc2, hws2, dinv, b2, Wf, bf, w1a, w1b)

    t1 = query_kernel(u, v, qs_r, qd_r)

    bq = 2048
    q_blocks = q_pad // bq
    out_full = pl.pallas_call(
        _tc_decoder,
        grid=(q_blocks,),
        in_specs=[
            pl.BlockSpec((bq, hid), lambda i: (i, 0)),
            pl.BlockSpec((bq,), lambda i: (i,)),
            pl.BlockSpec((bq,), lambda i: (i,)),
            pl.BlockSpec((hid,), lambda i: (0,)),
            pl.BlockSpec((hid,), lambda i: (0,)),
            pl.BlockSpec((hid,), lambda i: (0,)),
            pl.BlockSpec((hid, hid), lambda i: (0, 0)),
            pl.BlockSpec((hid,), lambda i: (0,)),
            pl.BlockSpec((hid,), lambda i: (0,)),
            pl.BlockSpec((1, 1), lambda i: (0, 0)),
        ],
        out_specs=pl.BlockSpec((bq,), lambda i: (i,)),
        out_shape=jax.ShapeDtypeStruct((q_pad,), jnp.float32),
    )(t1, idiff, spart, wci, wcs, bd1, w2p, b2p, w3v, b3)

    return out_full[:q]


# speedup vs baseline: 1.0028x; 1.0028x over previous
import functools

import jax
import jax.numpy as jnp
from jax import lax
from jax.experimental import pallas as pl
from jax.experimental.pallas import tpu as pltpu
from jax.experimental.pallas import tpu_sc as plsc

NC = 2
NS = 16
NW = NC * NS
L = 16
K = 128
H = 64
QK = 64


def _make_deg_kernel(n_pad, rt, qrt):
    @functools.partial(
        pl.kernel,
        out_type=(
            jax.ShapeDtypeStruct((NW, n_pad), jnp.float32),
            jax.ShapeDtypeStruct((NW, qrt, K), jnp.float32),
            jax.ShapeDtypeStruct((NW, qrt, K), jnp.float32),
        ),
        mesh=plsc.VectorSubcoreMesh(core_axis_name="c", subcore_axis_name="s"),
        scratch_types=[
            pltpu.VMEM((n_pad,), jnp.float32),
            pltpu.VMEM((rt, K), jnp.int32),
            pltpu.VMEM((qrt, K), jnp.int32),
            pltpu.VMEM((qrt, K), jnp.int32),
            pltpu.VMEM((n_pad,), jnp.int32),
            pltpu.VMEM((qrt, K), jnp.float32),
            pltpu.VMEM((qrt, K), jnp.float32),
        ],
        compiler_params=pltpu.CompilerParams(needs_layout_passes=False),
    )
    def deg_kernel(dstr, qsr, qdr, pkp, outd, idiff, spart,
                   deg_v, idx_v, qi_s, qi_d, pk_v, fb1, fb2):
        cid = lax.axis_index("c")
        sid = lax.axis_index("s")
        wid = cid * NS + sid
        ones = jnp.ones((L,), jnp.float32)
        pltpu.sync_copy(dstr.at[pl.ds(wid * rt, rt)], idx_v)
        pltpu.sync_copy(qsr.at[wid], qi_s)
        pltpu.sync_copy(qdr.at[wid], qi_d)
        pltpu.sync_copy(pkp, pk_v)

        def zero_body(i, _):
            deg_v[pl.ds(i * L, L)] = jnp.zeros((L,), jnp.float32)
            return 0
        lax.fori_loop(0, n_pad // L, zero_body, 0)

        def row_body(j, _):
            for g in range(K // L):
                dv = idx_v[j, pl.ds(g * L, L)]
                plsc.addupdate_scatter(deg_v, [dv], ones)
            return 0
        lax.fori_loop(0, rt, row_body, 0)
        pltpu.sync_copy(deg_v, outd.at[wid])

        def feat_body(j, _):
            for g in range(K // L):
                qsv = qi_s[j, pl.ds(g * L, L)]
                qdv = qi_d[j, pl.ds(g * L, L)]
                pvs = plsc.load_gather(pk_v, [qsv])
                pvd = plsc.load_gather(pk_v, [qdv])
                fb1[j, pl.ds(g * L, L)] = (
                    (pvd >> 7) - (pvs >> 7)).astype(jnp.float32)
                fb2[j, pl.ds(g * L, L)] = jnp.where(
                    (pvd & 127) == (pvs & 127),
                    jnp.float32(1.0), jnp.float32(0.0))
            return 0
        lax.fori_loop(0, qrt, feat_body, 0)
        pltpu.sync_copy(fb1, idiff.at[wid])
        pltpu.sync_copy(fb2, spart.at[wid])

    return deg_kernel


def _make_scatter_kernel(n_pad, rt):
    zr = n_pad // NS
    hrt = rt // 2

    @functools.partial(
        pl.kernel,
        out_type=jax.ShapeDtypeStruct((NC, n_pad, 128), jnp.float32),
        mesh=plsc.VectorSubcoreMesh(core_axis_name="c", subcore_axis_name="s"),
        scratch_types=[
            pltpu.VMEM_SHARED((n_pad, 128), jnp.float32),
            pltpu.VMEM((hrt, K), jnp.int32),
            pltpu.VMEM((hrt, K), jnp.int32),
            pltpu.VMEM((K, 128), jnp.float32),
            pltpu.VMEM((K, 128), jnp.float32),
            pltpu.VMEM((8, 128), jnp.float32),
            pltpu.SemaphoreType.DMA,
            pltpu.SemaphoreType.DMA,
        ],
    )
    def scatter_kernel(hws_h, srcr, dstr, accp_h,
                       asp, idx_s, idx_d, rows0, rows1, zbuf,
                       sem0, sem1):
        cid = lax.axis_index("c")
        sid = lax.axis_index("s")
        wid = cid * NS + sid

        def zb(i, _):
            for g in range(128 // L):
                zbuf[i, pl.ds(g * L, L)] = jnp.zeros((L,), jnp.float32)
            return 0
        lax.fori_loop(0, 8, zb, 0)

        if True:
            def zc(t, _):
                pltpu.sync_copy(zbuf, asp.at[pl.ds(sid * zr + t * 8, 8)])
                return 0
            lax.fori_loop(0, zr // 8, zc, 0)
            plsc.subcore_barrier()

            for h in range(2):
                pltpu.sync_copy(srcr.at[pl.ds(wid * rt + h * hrt, hrt)], idx_s)
                pltpu.sync_copy(dstr.at[pl.ds(wid * rt + h * hrt, hrt)], idx_d)
                pltpu.async_copy(hws_h.at[idx_s.at[0]], rows0, sem0)

                def pair_body(g, _):
                    j0 = 2 * g
                    pltpu.async_copy(hws_h.at[idx_s.at[j0 + 1]], rows1, sem1)
                    pltpu.make_async_copy(
                        hws_h.at[pl.ds(0, K)], rows0, sem0).wait()
                    pltpu.sync_copy(rows0, asp.at[idx_d.at[j0]], add=True)

                    @pl.when(g < hrt // 2 - 1)
                    def _():
                        pltpu.async_copy(hws_h.at[idx_s.at[j0 + 2]], rows0, sem0)
                    pltpu.make_async_copy(
                        hws_h.at[pl.ds(0, K)], rows1, sem1).wait()
                    pltpu.sync_copy(rows1, asp.at[idx_d.at[j0 + 1]], add=True)
                    return 0
                lax.fori_loop(0, hrt // 2, pair_body, 0)
            plsc.subcore_barrier()

            pltpu.sync_copy(asp.at[pl.ds(sid * zr, zr)],
                            accp_h.at[cid, pl.ds(sid * zr, zr)])

    return scatter_kernel


def _make_query_kernel(n_pad, q_pad, qrt):
    zr = n_pad // NS

    @functools.partial(
        pl.kernel,
        out_type=(
            jax.ShapeDtypeStruct((q_pad, 128), jnp.float32),
            jax.ShapeDtypeStruct((q_pad, 128), jnp.float32),
        ),
        mesh=plsc.VectorSubcoreMesh(core_axis_name="c", subcore_axis_name="s"),
        scratch_types=[
            pltpu.VMEM((qrt, K), jnp.int32),
            pltpu.VMEM((qrt, K), jnp.int32),
            pltpu.VMEM((K, 128), jnp.float32),
            pltpu.VMEM((K, 128), jnp.float32),
            pltpu.VMEM((K, 128), jnp.float32),
            pltpu.VMEM((K, 128), jnp.float32),
            pltpu.SemaphoreType.DMA,
            pltpu.SemaphoreType.DMA,
            pltpu.SemaphoreType.DMA,
            pltpu.SemaphoreType.DMA,
        ],
    )
    def query_kernel(u_h, v_h, qsr, qdr, zs, zd,
                     idx_s, idx_d, rs0, rs1, rd0, rd1,
                     ss0, ss1, sd0, sd1):
        cid = lax.axis_index("c")
        sid = lax.axis_index("s")
        wid = cid * NS + sid
        pltpu.sync_copy(qsr.at[wid], idx_s)
        pltpu.sync_copy(qdr.at[wid], idx_d)

        def isl(ib, j):
            return ib.at[j]

        pltpu.async_copy(u_h.at[isl(idx_s, 0)], rs0, ss0)
        pltpu.async_copy(v_h.at[isl(idx_d, 0)], rd0, sd0)

        def pair_body(g, _):
            j0 = 2 * g
            pltpu.async_copy(u_h.at[isl(idx_s, j0 + 1)], rs1, ss1)
            pltpu.async_copy(v_h.at[isl(idx_d, j0 + 1)], rd1, sd1)
            pltpu.make_async_copy(u_h.at[pl.ds(0, K)], rs0, ss0).wait()
            pltpu.sync_copy(rs0, zs.at[pl.ds((wid * qrt + j0) * K, K)])
            pltpu.make_async_copy(u_h.at[pl.ds(0, K)], rd0, sd0).wait()
            pltpu.sync_copy(rd0, zd.at[pl.ds((wid * qrt + j0) * K, K)])

            @pl.when(g < qrt // 2 - 1)
            def _():
                pltpu.async_copy(u_h.at[isl(idx_s, j0 + 2)], rs0, ss0)
                pltpu.async_copy(v_h.at[isl(idx_d, j0 + 2)], rd0, sd0)
            pltpu.make_async_copy(u_h.at[pl.ds(0, K)], rs1, ss1).wait()
            pltpu.sync_copy(rs1, zs.at[pl.ds((wid * qrt + j0 + 1) * K, K)])
            pltpu.make_async_copy(u_h.at[pl.ds(0, K)], rd1, sd1).wait()
            pltpu.sync_copy(rd1, zd.at[pl.ds((wid * qrt + j0 + 1) * K, K)])
            return 0
        lax.fori_loop(0, qrt // 2, pair_body, 0)

    return query_kernel


def _tc_first(degp_blk, x_blk, w_blk, dinv_blk, hws_blk):
    deg = jnp.sum(degp_blk[...], axis=0) + 1.0
    dinv = lax.rsqrt(deg)
    dinv_blk[...] = dinv
    hw = jnp.dot(x_blk[...], w_blk[...], preferred_element_type=jnp.float32)
    hws_blk[...] = hw * dinv[:, None]


def _tc_mid(accp_blk, hws_blk, dinv_blk, b_blk, w_blk, out_blk):
    acc = accp_blk[0] + accp_blk[1] + hws_blk[...]
    dinv = dinv_blk[...]
    h = jax.nn.relu(acc * dinv[:, None] + b_blk[...][None, :])
    out_blk[...] = jnp.dot(h, w_blk[...],
                           preferred_element_type=jnp.float32) * dinv[:, None]


def _tc_last(accp_blk, hws_blk, dinv_blk, b_blk, w_blk, bf_blk,
             w1a_blk, w1b_blk, u_blk, v_blk):
    acc = accp_blk[0] + accp_blk[1] + hws_blk[...]
    dinv = dinv_blk[...]
    h = jax.nn.relu(acc * dinv[:, None] + b_blk[...][None, :])
    z = jnp.dot(h, w_blk[...],
                preferred_element_type=jnp.float32) + bf_blk[...][None, :]
    u_blk[...] = jnp.dot(z, w1a_blk[...], preferred_element_type=jnp.float32)
    v_blk[...] = jnp.dot(z, w1b_blk[...], preferred_element_type=jnp.float32)


def _tc_decoder(zs_blk, zd_blk, idf_blk, spt_blk, wci, wcs, b1,
                w2, b2, w3, b3, out_blk):
    t = zs_blk[...] + zd_blk[...]
    t = t + idf_blk[...][:, None] * wci[...][None, :]
    t = t + spt_blk[...][:, None] * wcs[...][None, :]
    t = jax.nn.relu(t + b1[...][None, :])
    t = jax.nn.relu(jnp.dot(t, w2[...], preferred_element_type=jnp.float32)
                    + b2[...][None, :])
    logit = jnp.sum(t * w3[...][None, :], axis=1) + b3[0, 0]
    out_blk[...] = jax.nn.sigmoid(logit)


def kernel(x, edge_index, query_edges, int_values, partition_ids,
           W1, b1, W2, b2, Wf, bf, Wd1, bd1, Wd2, bd2, Wd3, bd3):
    n, f_in = x.shape
    e = edge_index.shape[1]
    q = query_edges.shape[1]
    hid = W1.shape[1]

    bn = 2048
    n_pad = ((n + bn - 1) // bn) * bn
    n_blocks = n_pad // bn

    rt = -(-e // (K * NW))
    rt = ((rt + 3) // 4) * 4
    e_pad = rt * NW * K
    src_r = jnp.pad(edge_index[0], (0, e_pad - e)).reshape(rt * NW, K)
    dst_r = jnp.pad(edge_index[1], (0, e_pad - e),
                    constant_values=n).reshape(rt * NW, K)

    qrt = -(-q // (K * NW))
    qrt = qrt + (qrt % 2)
    q_pad = qrt * NW * K
    qs_r = jnp.pad(query_edges[0], (0, q_pad - q)).reshape(NW, qrt, K)
    qd_r = jnp.pad(query_edges[1], (0, q_pad - q)).reshape(NW, qrt, K)
    pk_p = jnp.pad(int_values * 128 + partition_ids, (0, n_pad - n))

    deg_kernel = _make_deg_kernel(n_pad, rt, qrt)
    scatter_kernel = _make_scatter_kernel(n_pad, rt)
    query_kernel = _make_query_kernel(n_pad, q_pad, qrt)

    degp, idiff, spart = deg_kernel(dst_r, qs_r, qd_r, pk_p)
    idiff = idiff.reshape(q_pad)
    spart = spart.reshape(q_pad)

    grid_n = (n_blocks,)
    full_spec = pl.BlockSpec((bn, hid), lambda i: (i, 0))
    full_shape = jax.ShapeDtypeStruct((n_pad, hid), jnp.float32)
    accp_spec = pl.BlockSpec((NC, bn, hid), lambda i: (0, i, 0))

    dinv, hws1 = pl.pallas_call(
        _tc_first,
        grid=grid_n,
        in_specs=[
            pl.BlockSpec((NW, bn), lambda i: (0, i)),
            pl.BlockSpec((bn, f_in), lambda i: (i, 0)),
            pl.BlockSpec((f_in, hid), lambda i: (0, 0)),
        ],
        out_specs=[pl.BlockSpec((bn,), lambda i: (i,)), full_spec],
        out_shape=[jax.ShapeDtypeStruct((n_pad,), jnp.float32), full_shape],
    )(degp, x, W1)

    acc1 = scatter_kernel(hws1, src_r, dst_r)

    hws2 = pl.pallas_call(
        _tc_mid,
        grid=grid_n,
        in_specs=[
            accp_spec,
            full_spec,
            pl.BlockSpec((bn,), lambda i: (i,)),
            pl.BlockSpec((hid,), lambda i: (0,)),
            pl.BlockSpec((hid, hid), lambda i: (0, 0)),
        ],
        out_specs=full_spec,
        out_shape=full_shape,
    )(acc1, hws1, dinv, b1, W2)

    acc2 = scatter_kernel(hws2, src_r, dst_r)

    emb = Wf.shape[1]
    hid2 = Wd2.shape[1]
    w1a = Wd1[:emb]
    w1b = Wd1[emb:2 * emb]
    wci = Wd1[2 * emb]
    wcs = Wd1[2 * emb + 1]
    w2p = jnp.pad(Wd2, ((0, 0), (0, hid - hid2)))
    b2p = jnp.pad(bd2, (0, hid - hid2))
    w3v = jnp.pad(Wd3[:, 0], (0, hid - hid2))
    b3 = bd3.reshape(1, 1)

    u, v = pl.pallas_call(
        _tc_last,
        grid=grid_n,
        in_specs=[
            accp_spec,
            full_spec,
            pl.BlockSpec((bn,), lambda i: (i,)),
            pl.BlockSpec((hid,), lambda i: (0,)),
            pl.BlockSpec((hid, emb), lambda i: (0, 0)),
            pl.BlockSpec((emb,), lambda i: (0,)),
            pl.BlockSpec((emb, hid), lambda i: (0, 0)),
            pl.BlockSpec((emb, hid), lambda i: (0, 0)),
        ],
        out_specs=[full_spec, full_spec],
        out_shape=[full_shape, full_shape],
    )(acc2, hws2, dinv, b2, Wf, bf, w1a, w1b)

    zs, zd = query_kernel(u, v, qs_r, qd_r)

    bq = 2048
    q_blocks = q_pad // bq
    out_full = pl.pallas_call(
        _tc_decoder,
        grid=(q_blocks,),
        in_specs=[
            pl.BlockSpec((bq, hid), lambda i: (i, 0)),
            pl.BlockSpec((bq, hid), lambda i: (i, 0)),
            pl.BlockSpec((bq,), lambda i: (i,)),
            pl.BlockSpec((bq,), lambda i: (i,)),
            pl.BlockSpec((hid,), lambda i: (0,)),
            pl.BlockSpec((hid,), lambda i: (0,)),
            pl.BlockSpec((hid,), lambda i: (0,)),
            pl.BlockSpec((hid, hid), lambda i: (0, 0)),
            pl.BlockSpec((hid,), lambda i: (0,)),
            pl.BlockSpec((hid,), lambda i: (0,)),
            pl.BlockSpec((1, 1), lambda i: (0, 0)),
        ],
        out_specs=pl.BlockSpec((bq,), lambda i: (i,)),
        out_shape=jax.ShapeDtypeStruct((q_pad,), jnp.float32),
    )(zs, zd, idiff, spart, wci, wcs, bd1, w2p, b2p, w3v, b3)

    return out_full[:q]


# speedup vs baseline: 1.1878x; 1.1845x over previous
import functools

import jax
import jax.numpy as jnp
from jax import lax
from jax.experimental import pallas as pl
from jax.experimental.pallas import tpu as pltpu
from jax.experimental.pallas import tpu_sc as plsc

NC = 2
NS = 16
NW = NC * NS
L = 16
K = 128

_mesh = lambda: plsc.VectorSubcoreMesh(core_axis_name="c", subcore_axis_name="s")


def _make_deg_kernel(n_pad, rt):
    @functools.partial(
        pl.kernel,
        out_type=jax.ShapeDtypeStruct((NW, n_pad), jnp.float32),
        mesh=_mesh(),
        scratch_types=[
            pltpu.VMEM((n_pad,), jnp.float32),
            pltpu.VMEM((K,), jnp.int32),
        ],
        compiler_params=pltpu.CompilerParams(needs_layout_passes=False),
    )
    def deg_kernel(dstr, outd, deg_v, idx_v):
        cid = lax.axis_index("c")
        sid = lax.axis_index("s")
        wid = cid * NS + sid
        ones = jnp.ones((L,), jnp.float32)

        def zero_body(i, _):
            deg_v[pl.ds(i * L, L)] = jnp.zeros((L,), jnp.float32)
            return 0
        lax.fori_loop(0, n_pad // L, zero_body, 0)

        def row_body(j, _):
            pltpu.sync_copy(dstr.at[wid * rt + j], idx_v)
            def grp_body(g, _):
                dv = idx_v[pl.ds(g * L, L)]
                plsc.addupdate_scatter(deg_v, [dv], ones)
                return 0
            lax.fori_loop(0, K // L, grp_body, 0)
            return 0
        lax.fori_loop(0, rt, row_body, 0)

        pltpu.sync_copy(deg_v, outd.at[wid])

    return deg_kernel


def _make_scatter_kernel(n, n_pad, rt):
    zr = n_pad // NS

    @functools.partial(
        pl.kernel,
        out_type=jax.ShapeDtypeStruct((NC, n_pad, 128), jnp.float32),
        mesh=_mesh(),
        scratch_types=[
            pltpu.VMEM_SHARED((n_pad, 128), jnp.float32),
            pltpu.VMEM((K,), jnp.int32),
            pltpu.VMEM((K,), jnp.int32),
            pltpu.VMEM((K, 128), jnp.float32),
            pltpu.VMEM((L, 128), jnp.float32),
            pltpu.SemaphoreType.DMA,
        ],
    )
    def scatter_kernel(hws, srcr, dstr, accp, acc, idx_s, idx_d, rows, zbuf, sem):
        cid = lax.axis_index("c")
        sid = lax.axis_index("s")
        wid = cid * NS + sid

        def zb(i, _):
            def zg(g, _):
                zbuf[i, pl.ds(g * L, L)] = jnp.zeros((L,), jnp.float32)
                return 0
            lax.fori_loop(0, 128 // L, zg, 0)
            return 0
        lax.fori_loop(0, L, zb, 0)

        def zc(t, _):
            pltpu.sync_copy(zbuf, acc.at[pl.ds(sid * zr + t * L, L)])
            return 0
        lax.fori_loop(0, zr // L, zc, 0)
        plsc.subcore_barrier()

        def row_body(j, _):
            r = wid * rt + j
            pltpu.sync_copy(srcr.at[r], idx_s)
            pltpu.sync_copy(dstr.at[r], idx_d)
            pltpu.async_copy(hws.at[idx_s], rows, sem).wait()
            pltpu.sync_copy(rows, acc.at[idx_d], add=True)
            return 0
        lax.fori_loop(0, rt, row_body, 0)
        plsc.subcore_barrier()

        pltpu.sync_copy(acc.at[pl.ds(sid * zr, zr)],
                        accp.at[cid, pl.ds(sid * zr, zr)])

    return scatter_kernel


def _make_query_kernel(n_pad, q_pad, qrt):
    @functools.partial(
        pl.kernel,
        out_type=(
            jax.ShapeDtypeStruct((q_pad, 128), jnp.float32),
            jax.ShapeDtypeStruct((q_pad, 128), jnp.float32),
            jax.ShapeDtypeStruct((q_pad,), jnp.float32),
            jax.ShapeDtypeStruct((q_pad,), jnp.float32),
        ),
        mesh=_mesh(),
        scratch_types=[
            pltpu.VMEM((K,), jnp.int32),
            pltpu.VMEM((K,), jnp.int32),
            pltpu.VMEM((K, 128), jnp.float32),
            pltpu.VMEM((K, 128), jnp.float32),
            pltpu.VMEM((n_pad,), jnp.int32),
            pltpu.VMEM((n_pad,), jnp.int32),
            pltpu.VMEM((K,), jnp.float32),
            pltpu.VMEM((K,), jnp.float32),
            pltpu.SemaphoreType.DMA,
            pltpu.SemaphoreType.DMA,
        ],
        compiler_params=pltpu.CompilerParams(needs_layout_passes=False),
    )
    def query_kernel(z, qsr, qdr, ivp, pidp, zs, zd, idiff, spart,
                     idx_s, idx_d, rows_s, rows_d, iv_v, pid_v, fb1, fb2,
                     sem1, sem2):
        cid = lax.axis_index("c")
        sid = lax.axis_index("s")
        wid = cid * NS + sid
        pltpu.sync_copy(ivp, iv_v)
        pltpu.sync_copy(pidp, pid_v)

        def row_body(j, _):
            r = wid * qrt + j
            pltpu.sync_copy(qsr.at[r], idx_s)
            pltpu.sync_copy(qdr.at[r], idx_d)
            cs = pltpu.async_copy(z.at[idx_s], rows_s, sem1)
            cd = pltpu.async_copy(z.at[idx_d], rows_d, sem2)

            def grp_body(g, _):
                qsv = idx_s[pl.ds(g * L, L)]
                qdv = idx_d[pl.ds(g * L, L)]
                ivs = plsc.load_gather(iv_v, [qsv])
                ivd = plsc.load_gather(iv_v, [qdv])
                fb1[pl.ds(g * L, L)] = (ivd - ivs).astype(jnp.float32)
                ps = plsc.load_gather(pid_v, [qsv])
                pd = plsc.load_gather(pid_v, [qdv])
                fb2[pl.ds(g * L, L)] = jnp.where(
                    ps == pd, jnp.float32(1.0), jnp.float32(0.0))
                return 0
            lax.fori_loop(0, K // L, grp_body, 0)

            pltpu.sync_copy(fb1, idiff.at[pl.ds(r * K, K)])
            pltpu.sync_copy(fb2, spart.at[pl.ds(r * K, K)])
            cs.wait()
            pltpu.sync_copy(rows_s, zs.at[pl.ds(r * K, K)])
            cd.wait()
            pltpu.sync_copy(rows_d, zd.at[pl.ds(r * K, K)])
            return 0
        lax.fori_loop(0, qrt, row_body, 0)

    return query_kernel


def _tc_first(degp_blk, x_blk, w_blk, dinv_blk, hws_blk):
    deg = jnp.sum(degp_blk[...], axis=0) + 1.0
    dinv = lax.rsqrt(deg)
    dinv_blk[...] = dinv
    hw = jnp.dot(x_blk[...], w_blk[...], preferred_element_type=jnp.float32)
    hws_blk[...] = hw * dinv[:, None]


def _tc_mid(accp_blk, hws_blk, dinv_blk, b_blk, w_blk, out_blk):
    acc = accp_blk[0] + accp_blk[1] + hws_blk[...]
    dinv = dinv_blk[...]
    h = jax.nn.relu(acc * dinv[:, None] + b_blk[...][None, :])
    out_blk[...] = jnp.dot(h, w_blk[...],
                           preferred_element_type=jnp.float32) * dinv[:, None]


def _tc_last(accp_blk, hws_blk, dinv_blk, b_blk, w_blk, bf_blk, z_blk):
    acc = accp_blk[0] + accp_blk[1] + hws_blk[...]
    dinv = dinv_blk[...]
    h = jax.nn.relu(acc * dinv[:, None] + b_blk[...][None, :])
    z_blk[...] = jnp.dot(h, w_blk[...],
                         preferred_element_type=jnp.float32) + bf_blk[...][None, :]


def _tc_decoder(zs_blk, zd_blk, idf_blk, spt_blk, w1a, w1b, wci, wcs, b1,
                w2, b2, w3, b3, out_blk):
    t = jnp.dot(zs_blk[...], w1a[...], preferred_element_type=jnp.float32)
    t = t + jnp.dot(zd_blk[...], w1b[...], preferred_element_type=jnp.float32)
    t = t + idf_blk[...][:, None] * wci[...][None, :]
    t = t + spt_blk[...][:, None] * wcs[...][None, :]
    t = jax.nn.relu(t + b1[...][None, :])
    t = jax.nn.relu(jnp.dot(t, w2[...], preferred_element_type=jnp.float32)
                    + b2[...][None, :])
    logit = jnp.sum(t * w3[...][None, :], axis=1) + b3[0, 0]
    out_blk[...] = jax.nn.sigmoid(logit)


def kernel(x, edge_index, query_edges, int_values, partition_ids,
           W1, b1, W2, b2, Wf, bf, Wd1, bd1, Wd2, bd2, Wd3, bd3):
    n, f_in = x.shape
    e = edge_index.shape[1]
    q = query_edges.shape[1]
    hid = W1.shape[1]

    bn = 2048
    n_pad = ((n + bn - 1) // bn) * bn
    n_blocks = n_pad // bn

    rt = -(-e // (K * NW))
    e_pad = rt * NW * K
    src_r = jnp.pad(edge_index[0], (0, e_pad - e)).reshape(rt * NW, K)
    dst_r = jnp.pad(edge_index[1], (0, e_pad - e),
                    constant_values=n).reshape(rt * NW, K)

    qrt = -(-q // (K * NW))
    q_pad = qrt * NW * K
    qs_r = jnp.pad(query_edges[0], (0, q_pad - q)).reshape(qrt * NW, K)
    qd_r = jnp.pad(query_edges[1], (0, q_pad - q)).reshape(qrt * NW, K)
    iv_p = jnp.pad(int_values, (0, n_pad - n))
    pid_p = jnp.pad(partition_ids, (0, n_pad - n))

    deg_kernel = _make_deg_kernel(n_pad, rt)
    scatter_kernel = _make_scatter_kernel(n, n_pad, rt)
    query_kernel = _make_query_kernel(n_pad, q_pad, qrt)

    degp = deg_kernel(dst_r)

    grid_n = (n_blocks,)
    dinv, hws1 = pl.pallas_call(
        _tc_first,
        grid=grid_n,
        in_specs=[
            pl.BlockSpec((NW, bn), lambda i: (0, i)),
            pl.BlockSpec((bn, f_in), lambda i: (i, 0)),
            pl.BlockSpec((f_in, hid), lambda i: (0, 0)),
        ],
        out_specs=[
            pl.BlockSpec((bn,), lambda i: (i,)),
            pl.BlockSpec((bn, hid), lambda i: (i, 0)),
        ],
        out_shape=[
            jax.ShapeDtypeStruct((n_pad,), jnp.float32),
            jax.ShapeDtypeStruct((n, hid), jnp.float32),
        ],
    )(degp, x, W1)

    accp1 = scatter_kernel(hws1, src_r, dst_r)

    hws2 = pl.pallas_call(
        _tc_mid,
        grid=grid_n,
        in_specs=[
            pl.BlockSpec((NC, bn, hid), lambda i: (0, i, 0)),
            pl.BlockSpec((bn, hid), lambda i: (i, 0)),
            pl.BlockSpec((bn,), lambda i: (i,)),
            pl.BlockSpec((hid,), lambda i: (0,)),
            pl.BlockSpec((hid, hid), lambda i: (0, 0)),
        ],
        out_specs=pl.BlockSpec((bn, hid), lambda i: (i, 0)),
        out_shape=jax.ShapeDtypeStruct((n, hid), jnp.float32),
    )(accp1, hws1, dinv, b1, W2)

    accp2 = scatter_kernel(hws2, src_r, dst_r)

    emb = Wf.shape[1]
    z = pl.pallas_call(
        _tc_last,
        grid=grid_n,
        in_specs=[
            pl.BlockSpec((NC, bn, hid), lambda i: (0, i, 0)),
            pl.BlockSpec((bn, hid), lambda i: (i, 0)),
            pl.BlockSpec((bn,), lambda i: (i,)),
            pl.BlockSpec((hid,), lambda i: (0,)),
            pl.BlockSpec((hid, emb), lambda i: (0, 0)),
            pl.BlockSpec((emb,), lambda i: (0,)),
        ],
        out_specs=pl.BlockSpec((bn, emb), lambda i: (i, 0)),
        out_shape=jax.ShapeDtypeStruct((n, emb), jnp.float32),
    )(accp2, hws2, dinv, b2, Wf, bf)

    zs, zd, idiff, spart = query_kernel(z, qs_r, qd_r, iv_p, pid_p)

    hid2 = Wd2.shape[1]
    w1a = Wd1[:emb]
    w1b = Wd1[emb:2 * emb]
    wci = Wd1[2 * emb]
    wcs = Wd1[2 * emb + 1]
    w2p = jnp.pad(Wd2, ((0, 0), (0, hid - hid2)))
    b2p = jnp.pad(bd2, (0, hid - hid2))
    w3v = jnp.pad(Wd3[:, 0], (0, hid - hid2))
    b3 = bd3.reshape(1, 1)

    bq = 2048
    q_blocks = q_pad // bq
    out_full = pl.pallas_call(
        _tc_decoder,
        grid=(q_blocks,),
        in_specs=[
            pl.BlockSpec((bq, emb), lambda i: (i, 0)),
            pl.BlockSpec((bq, emb), lambda i: (i, 0)),
            pl.BlockSpec((bq,), lambda i: (i,)),
            pl.BlockSpec((bq,), lambda i: (i,)),
            pl.BlockSpec((emb, hid), lambda i: (0, 0)),
            pl.BlockSpec((emb, hid), lambda i: (0, 0)),
            pl.BlockSpec((hid,), lambda i: (0,)),
            pl.BlockSpec((hid,), lambda i: (0,)),
            pl.BlockSpec((hid,), lambda i: (0,)),
            pl.BlockSpec((hid, hid), lambda i: (0, 0)),
            pl.BlockSpec((hid,), lambda i: (0,)),
            pl.BlockSpec((hid,), lambda i: (0,)),
            pl.BlockSpec((1, 1), lambda i: (0, 0)),
        ],
        out_specs=pl.BlockSpec((bq,), lambda i: (i,)),
        out_shape=jax.ShapeDtypeStruct((q_pad,), jnp.float32),
    )(zs, zd, idiff, spart, w1a, w1b, wci, wcs, bd1, w2p, b2p, w3v, b3)

    return out_full[:q]
